# baseline (speedup 1.0000x reference)
import functools

import jax
import jax.numpy as jnp
from jax import lax
from jax.experimental import pallas as pl
from jax.experimental.pallas import tpu as pltpu


def _decode_body(scale_ref, x_ref, o_ref, *, nA, nh, nw):
    x = x_ref[0]
    nc = nA * 5
    c = lax.broadcasted_iota(jnp.int32, (nc, 1, 1), 0) % 5
    gx = lax.broadcasted_iota(jnp.int32, (1, nh, nw), 2).astype(jnp.float32)
    gy = lax.broadcasted_iota(jnp.int32, (1, nh, nw), 1).astype(jnp.float32)
    add = jnp.where(c == 0, gx, 0.0) + jnp.where(c == 1, gy, 0.0)
    is_sig = (c != 2) & (c != 3)
    val = jnp.where(is_sig, jax.nn.sigmoid(x), jnp.exp(x))
    val = (val + add) * scale_ref[0].reshape(nc, 1, 1)
    valf = val.reshape(nc, nh * nw)
    eye = jnp.eye(5, dtype=jnp.float32)
    hw = nh * nw
    for a in range(nA):
        seg = valf[5 * a:5 * a + 5, :]
        t = lax.dot_general(seg, eye, (((0,), (0,)), ((), ())),
                            preferred_element_type=jnp.float32,
                            precision=lax.Precision.HIGHEST)
        o_ref[0, a * hw:(a + 1) * hw, :] = t


def kernel(out, size, infer, anchors):
    nb, nc, nh, nw = out.shape
    nA = nc // 5
    scale = jnp.stack([
        jnp.full((nA,), 1.0 / nw, jnp.float32),
        jnp.full((nA,), 1.0 / nh, jnp.float32),
        anchors[:, 0],
        anchors[:, 1],
        jnp.ones((nA,), jnp.float32),
    ], axis=1).reshape(1, nc)

    fn = pl.pallas_call(
        functools.partial(_decode_body, nA=nA, nh=nh, nw=nw),
        grid=(nb,),
        in_specs=[
            pl.BlockSpec((1, nc), lambda b: (0, 0)),
            pl.BlockSpec((1, nc, nh, nw), lambda b: (b, 0, 0, 0)),
        ],
        out_specs=pl.BlockSpec((1, nA * nh * nw, 5), lambda b: (b, 0, 0)),
        out_shape=jax.ShapeDtypeStruct((nb, nA * nh * nw, 5), jnp.float32),
    )
    return fn(scale, out)


# speedup vs baseline: 3.5353x; 3.5353x over previous
import functools

import jax
import jax.numpy as jnp
from jax import lax
from jax.experimental import pallas as pl
from jax.experimental.pallas import tpu as pltpu


def _decode_body(scale_ref, x_ref, o_ref, *, nA, nh, nw):
    j = lax.broadcasted_iota(jnp.int32, (nh, 5 * nw), 1)
    jc = j % 5
    jx = j // 5
    gxn = lax.broadcasted_iota(jnp.int32, (nh, nw), 1).astype(jnp.float32) * (1.0 / nw)
    gyn = lax.broadcasted_iota(jnp.int32, (nh, nw), 0).astype(jnp.float32) * (1.0 / nh)

    def sig(v):
        return 1.0 / (1.0 + jnp.exp(-v))

    for a in range(nA):
        x = x_ref[0, 5 * a]
        planes = [
            sig(x_ref[0, 5 * a + 0]) * (1.0 / nw) + gxn,
            sig(x_ref[0, 5 * a + 1]) * (1.0 / nh) + gyn,
            jnp.exp(x_ref[0, 5 * a + 2]) * scale_ref[0, 5 * a + 2],
            jnp.exp(x_ref[0, 5 * a + 3]) * scale_ref[0, 5 * a + 3],
            sig(x_ref[0, 5 * a + 4]),
        ]
        acc = jnp.take_along_axis(planes[0], jx, axis=1)
        for cc in range(1, 5):
            acc = jnp.where(jc == cc, jnp.take_along_axis(planes[cc], jx, axis=1), acc)
        o_ref[0, a * nh:(a + 1) * nh, :] = acc


def kernel(out, size, infer, anchors):
    nb, nc, nh, nw = out.shape
    nA = nc // 5
    scale = jnp.stack([
        jnp.full((nA,), 1.0 / nw, jnp.float32),
        jnp.full((nA,), 1.0 / nh, jnp.float32),
        anchors[:, 0],
        anchors[:, 1],
        jnp.ones((nA,), jnp.float32),
    ], axis=1).reshape(1, nc)

    fn = pl.pallas_call(
        functools.partial(_decode_body, nA=nA, nh=nh, nw=nw),
        grid=(nb,),
        in_specs=[
            pl.BlockSpec(memory_space=pltpu.SMEM),
            pl.BlockSpec((1, nc, nh, nw), lambda b: (b, 0, 0, 0)),
        ],
        out_specs=pl.BlockSpec((1, nA * nh, nw * 5), lambda b: (b, 0, 0)),
        out_shape=jax.ShapeDtypeStruct((nb, nA * nh, nw * 5), jnp.float32),
        compiler_params=pltpu.CompilerParams(
            dimension_semantics=("parallel",),
        ),
    )
    return fn(scale, out).reshape(nb, nA * nh * nw, 5)


# speedup vs baseline: 4.2930x; 1.2143x over previous
import functools

import jax
import jax.numpy as jnp
from jax import lax
from jax.experimental import pallas as pl
from jax.experimental.pallas import tpu as pltpu


def _decode_body(scale_ref, x_ref, o_ref, *, nA, nh, nw, bblk):
    S = 8
    W = 5 * nw
    nJ = (W + 127) // 128
    jxs, msks = [], []
    for J in range(nJ):
        cw = min(128, W - 128 * J)
        j = lax.broadcasted_iota(jnp.int32, (S, cw), 1) + 128 * J
        jxs.append(j // 5)
        msks.append([j % 5 == cc for cc in range(1, 5)])
    gxn = lax.broadcasted_iota(jnp.int32, (S, nw), 1).astype(jnp.float32) * (1.0 / nw)
    gy0 = lax.broadcasted_iota(jnp.int32, (S, nw), 0).astype(jnp.float32) * (1.0 / nh)

    def sig(v):
        return 1.0 / (1.0 + jnp.exp(-v))

    for bi in range(bblk):
        for a in range(nA):
            aw = scale_ref[0, 5 * a + 2]
            ah = scale_ref[0, 5 * a + 3]
            for y0 in range(0, nh, S):
                row = pl.ds(y0, S)
                planes = [
                    sig(x_ref[bi, 5 * a + 0, row, :]) * (1.0 / nw) + gxn,
                    sig(x_ref[bi, 5 * a + 1, row, :]) * (1.0 / nh) + (gy0 + (y0 / nh)),
                    jnp.exp(x_ref[bi, 5 * a + 2, row, :]) * aw,
                    jnp.exp(x_ref[bi, 5 * a + 3, row, :]) * ah,
                    sig(x_ref[bi, 5 * a + 4, row, :]),
                ]
                for J in range(nJ):
                    cw = jxs[J].shape[1]
                    acc = jnp.take_along_axis(planes[0], jxs[J], axis=1)
                    for cc in range(1, 5):
                        acc = jnp.where(msks[J][cc - 1],
                                        jnp.take_along_axis(planes[cc], jxs[J], axis=1),
                                        acc)
                    o_ref[bi, pl.ds(a * nh + y0, S), pl.ds(128 * J, cw)] = acc


def kernel(out, size, infer, anchors):
    nb, nc, nh, nw = out.shape
    nA = nc // 5
    scale = jnp.stack([
        jnp.full((nA,), 1.0 / nw, jnp.float32),
        jnp.full((nA,), 1.0 / nh, jnp.float32),
        anchors[:, 0],
        anchors[:, 1],
        jnp.ones((nA,), jnp.float32),
    ], axis=1).reshape(1, nc)

    bblk = 4
    fn = pl.pallas_call(
        functools.partial(_decode_body, nA=nA, nh=nh, nw=nw, bblk=bblk),
        grid=(nb // bblk,),
        in_specs=[
            pl.BlockSpec(memory_space=pltpu.SMEM),
            pl.BlockSpec((bblk, nc, nh, nw), lambda b: (b, 0, 0, 0)),
        ],
        out_specs=pl.BlockSpec((bblk, nA * nh, nw * 5), lambda b: (b, 0, 0)),
        out_shape=jax.ShapeDtypeStruct((nb, nA * nh, nw * 5), jnp.float32),
        compiler_params=pltpu.CompilerParams(
            dimension_semantics=("parallel",),
        ),
    )
    return fn(scale, out).reshape(nb, nA * nh * nw, 5)


# speedup vs baseline: 10.6012x; 2.4694x over previous
import functools

import jax
import jax.numpy as jnp
from jax import lax
from jax.experimental import pallas as pl
from jax.experimental.pallas import tpu as pltpu


def _decode_body(scale_ref, gx_ref, gy_ref, x_ref, o_ref, *, nA, hw):
    a = pl.program_id(1)

    def sig(v):
        return 1.0 / (1.0 + jnp.exp(-v))

    inv_w = scale_ref[0, 0]
    inv_h = scale_ref[0, 1]
    aw = scale_ref[a, 2]
    ah = scale_ref[a, 3]
    gx = gx_ref[0]
    gy = gy_ref[0]
    o_ref[0, 0, 0] = sig(x_ref[0]) * inv_w + gx
    o_ref[1, 0, 0] = sig(x_ref[1]) * inv_h + gy
    o_ref[2, 0, 0] = jnp.exp(x_ref[2]) * aw
    o_ref[3, 0, 0] = jnp.exp(x_ref[3]) * ah
    o_ref[4, 0, 0] = sig(x_ref[4])


def kernel(out, size, infer, anchors):
    nb, nc, nh, nw = out.shape
    nA = nc // 5
    hw = nh * nw
    BT = 8

    xt = out.transpose(1, 0, 2, 3).reshape(nc, nb, hw)

    scale = jnp.stack([
        jnp.full((nA,), 1.0 / nw, jnp.float32),
        jnp.full((nA,), 1.0 / nh, jnp.float32),
        anchors[:, 0],
        anchors[:, 1],
    ], axis=1)
    cell = jnp.arange(hw, dtype=jnp.int32)
    gx = ((cell % nw).astype(jnp.float32) * (1.0 / nw)).reshape(1, hw)
    gy = ((cell // nw).astype(jnp.float32) * (1.0 / nh)).reshape(1, hw)

    fn = pl.pallas_call(
        functools.partial(_decode_body, nA=nA, hw=hw),
        grid=(nb // BT, nA),
        in_specs=[
            pl.BlockSpec(memory_space=pltpu.SMEM),
            pl.BlockSpec((1, hw), lambda bt, a: (0, 0)),
            pl.BlockSpec((1, hw), lambda bt, a: (0, 0)),
            pl.BlockSpec((5, BT, hw), lambda bt, a: (a, bt, 0)),
        ],
        out_specs=pl.BlockSpec((5, 1, 1, BT, hw), lambda bt, a: (0, bt, a, 0, 0)),
        out_shape=jax.ShapeDtypeStruct((5, nb // BT, nA, BT, hw), jnp.float32),
        compiler_params=pltpu.CompilerParams(
            dimension_semantics=("parallel", "parallel"),
        ),
    )
    res = fn(scale, gx, gy, xt)
    return res.transpose(1, 3, 2, 4, 0).reshape(nb, nA * hw, 5)


# speedup vs baseline: 25.8149x; 2.4351x over previous
import functools

import jax
import jax.numpy as jnp
from jax import lax
from jax.experimental import pallas as pl
from jax.experimental.pallas import tpu as pltpu


def _decode_body(scale_ref, x_ref, o_ref, *, nA, nh, nw):
    a = pl.program_id(1)

    def sig(v):
        return 1.0 / (1.0 + jnp.exp(-v))

    inv_w = scale_ref[0, 0]
    inv_h = scale_ref[0, 1]
    aw = scale_ref[a, 2]
    ah = scale_ref[a, 3]
    BT = x_ref.shape[0]
    hw = nh * nw
    gx = lax.broadcasted_iota(jnp.int32, (nh, nw), 1).astype(jnp.float32) * (1.0 / nw)
    gy = lax.broadcasted_iota(jnp.int32, (nh, nw), 0).astype(jnp.float32) * (1.0 / nh)
    o_ref[0, 0, 0] = (sig(x_ref[:, 0]) * inv_w + gx).reshape(BT, hw)
    o_ref[1, 0, 0] = (sig(x_ref[:, 1]) * inv_h + gy).reshape(BT, hw)
    o_ref[2, 0, 0] = (jnp.exp(x_ref[:, 2]) * aw).reshape(BT, hw)
    o_ref[3, 0, 0] = (jnp.exp(x_ref[:, 3]) * ah).reshape(BT, hw)
    o_ref[4, 0, 0] = sig(x_ref[:, 4]).reshape(BT, hw)


def kernel(out, size, infer, anchors):
    nb, nc, nh, nw = out.shape
    nA = nc // 5
    hw = nh * nw
    BT = 8

    scale = jnp.stack([
        jnp.full((nA,), 1.0 / nw, jnp.float32),
        jnp.full((nA,), 1.0 / nh, jnp.float32),
        anchors[:, 0],
        anchors[:, 1],
    ], axis=1)

    fn = pl.pallas_call(
        functools.partial(_decode_body, nA=nA, nh=nh, nw=nw),
        grid=(nb // BT, nA),
        in_specs=[
            pl.BlockSpec(memory_space=pltpu.SMEM),
            pl.BlockSpec((BT, 5, nh, nw), lambda bt, a: (bt, a, 0, 0)),
        ],
        out_specs=pl.BlockSpec((5, 1, 1, BT, hw), lambda bt, a: (0, bt, a, 0, 0)),
        out_shape=jax.ShapeDtypeStruct((5, nb // BT, nA, BT, hw), jnp.float32),
        compiler_params=pltpu.CompilerParams(
            dimension_semantics=("parallel", "parallel"),
        ),
    )
    res = fn(scale, out)
    return res.transpose(1, 3, 2, 4, 0).reshape(nb, nA * hw, 5)


# speedup vs baseline: 26.3891x; 1.0222x over previous
import functools

import jax
import jax.numpy as jnp
from jax import lax
from jax.experimental import pallas as pl
from jax.experimental.pallas import tpu as pltpu


def _decode_body(scale_ref, gx_ref, gy_ref, x_ref, o_ref, *, nA, nh, nw):
    a = pl.program_id(1)

    def sig(v):
        return 1.0 / (1.0 + jnp.exp(-v))

    inv_w = scale_ref[0, 0]
    inv_h = scale_ref[0, 1]
    aw = scale_ref[a, 2]
    ah = scale_ref[a, 3]
    BT = x_ref.shape[0]
    hw = nh * nw
    flat = [x_ref[:, c].reshape(BT, hw) for c in range(5)]
    o_ref[0, 0, 0] = sig(flat[0]) * inv_w + gx_ref[0]
    o_ref[1, 0, 0] = sig(flat[1]) * inv_h + gy_ref[0]
    o_ref[2, 0, 0] = jnp.exp(flat[2]) * aw
    o_ref[3, 0, 0] = jnp.exp(flat[3]) * ah
    o_ref[4, 0, 0] = sig(flat[4])


def kernel(out, size, infer, anchors):
    nb, nc, nh, nw = out.shape
    nA = nc // 5
    hw = nh * nw
    BT = 8

    scale = jnp.stack([
        jnp.full((nA,), 1.0 / nw, jnp.float32),
        jnp.full((nA,), 1.0 / nh, jnp.float32),
        anchors[:, 0],
        anchors[:, 1],
    ], axis=1)
    cell = jnp.arange(hw, dtype=jnp.int32)
    gx = ((cell % nw).astype(jnp.float32) * (1.0 / nw)).reshape(1, hw)
    gy = ((cell // nw).astype(jnp.float32) * (1.0 / nh)).reshape(1, hw)

    fn = pl.pallas_call(
        functools.partial(_decode_body, nA=nA, nh=nh, nw=nw),
        grid=(nb // BT, nA),
        in_specs=[
            pl.BlockSpec(memory_space=pltpu.SMEM),
            pl.BlockSpec((1, hw), lambda bt, a: (0, 0)),
            pl.BlockSpec((1, hw), lambda bt, a: (0, 0)),
            pl.BlockSpec((BT, 5, nh, nw), lambda bt, a: (bt, a, 0, 0)),
        ],
        out_specs=pl.BlockSpec((5, 1, 1, BT, hw), lambda bt, a: (0, bt, a, 0, 0)),
        out_shape=jax.ShapeDtypeStruct((5, nb // BT, nA, BT, hw), jnp.float32),
        compiler_params=pltpu.CompilerParams(
            dimension_semantics=("parallel", "parallel"),
        ),
    )
    res = fn(scale, gx, gy, out)
    return res.transpose(1, 3, 2, 4, 0).reshape(nb, nA * hw, 5)
